# baseline (speedup 1.0000x reference)
import functools

import jax
import jax.numpy as jnp
from jax import lax
from jax.experimental import pallas as pl
from jax.experimental.pallas import tpu as pltpu
from jax.experimental.pallas import tpu_sc as plsc

_NBUF = 3
_GTILES = 2
_TILE = 128


@functools.partial(jax.jit, static_argnames=("bsz", "n", "d", "nb", "g"))
def _sc_gather(table, idx2d, bsz, n, d, nb, g):
    tile = _TILE
    full_tiles = n // tile
    pairs = full_tiles // g
    rem_tiles = full_tiles % g
    spec_w = rem_tiles * tile + n % tile
    ch = g * tile
    mesh = plsc.VectorSubcoreMesh(core_axis_name="c", subcore_axis_name="s")
    info = plsc.get_sparse_core_info()
    nw = info.num_cores * info.num_subcores

    units = bsz * pairs
    u_per_w = -(-units // nw)
    w_tiles = ((u_per_w + bsz - 1) // bsz + 2) * g
    w_cols = w_tiles * tile
    assert spec_w % 16 == 0 and w_tiles <= full_tiles and bsz <= nw

    @functools.partial(
        pl.kernel,
        out_type=jax.ShapeDtypeStruct((bsz * n, d), table.dtype),
        mesh=mesh,
        scratch_types=(
            [pltpu.VMEM((bsz, w_cols), jnp.int32)]
            + ([pltpu.VMEM((bsz, spec_w), jnp.int32)] if spec_w else [])
            + [pltpu.VMEM((ch,), jnp.int32) for _ in range(nb)]
            + [pltpu.VMEM((ch, d), table.dtype) for _ in range(nb)]
            + [pltpu.SemaphoreType.DMA for _ in range(2 * nb)]
        ),
    )
    def k(table_hbm, idx_hbm, out_hbm, wdw_v, *scr):
        if spec_w:
            spec_v, scr = scr[0], scr[1:]
        lsts, rows = scr[0:nb], scr[nb:2 * nb]
        semg, sems = scr[2 * nb:3 * nb], scr[3 * nb:]
        bufs = tuple(zip(lsts, rows, semg, sems))
        wid = lax.axis_index("s") * info.num_cores + lax.axis_index("c")
        u0 = (wid * units) // nw
        u1 = ((wid + 1) * units) // nw
        t0 = jnp.minimum((u0 // bsz) * g, full_tiles - w_tiles)

        def drain_scatter(rows_v, sem):
            pltpu.make_async_copy(
                rows_v, out_hbm.at[pl.ds(0, ch)], sem).wait()

        pltpu.sync_copy(
            idx_hbm.at[:, pl.ds(pl.multiple_of(t0 * tile, tile), w_cols)],
            wdw_v)

        if spec_w:
            @pl.when(wid < bsz)
            def _():
                base_col = (full_tiles - rem_tiles) * tile
                pltpu.sync_copy(
                    idx_hbm.at[:, pl.ds(base_col, spec_w)], spec_v)
                lst_v, rows_v, sg, ss = bufs[0]
                b = wid
                for j in range(spec_w // 16):
                    sl = pl.ds(j * 16, 16)
                    lst_v[sl] = spec_v[b, pl.ds(j * 16, 16)] + b * n
                pltpu.async_copy(
                    table_hbm.at[lst_v.at[pl.ds(0, spec_w)]],
                    rows_v.at[pl.ds(0, spec_w)], sg).wait()
                pltpu.sync_copy(
                    rows_v.at[pl.ds(0, spec_w)],
                    out_hbm.at[pl.ds(b * n + base_col, spec_w)])

        kmax = u_per_w + 1
        kmax_r = kmax + (-kmax) % nb

        @pl.loop(0, kmax_r, step=nb)
        def _(i):
            for par in range(nb):
                k_it = i + par
                u = u0 + k_it
                plst_v, prows_v, psg, pss = bufs[(par - 1) % nb]
                lst_v, rows_v, sg, ss = bufs[par]

                @pl.when(u < u1)
                def _():
                    p = u // bsz
                    b = u % bsz
                    col = (p * g - t0) * tile
                    boff = b * n
                    for j in range(ch // 16):
                        sl = pl.ds(j * 16, 16)
                        lst_v[sl] = wdw_v[b, pl.ds(col + j * 16, 16)] + boff

                    @pl.when(k_it >= nb)
                    def _():
                        drain_scatter(rows_v, ss)

                    pltpu.async_copy(table_hbm.at[lst_v], rows_v, sg)

                @pl.when((k_it >= 1) & (u0 + k_it - 1 < u1))
                def _():
                    pu = u0 + k_it - 1
                    pltpu.make_async_copy(
                        table_hbm.at[plst_v], prows_v, psg).wait()
                    pltpu.async_copy(
                        prows_v,
                        out_hbm.at[pl.ds(
                            (pu % bsz) * n + (pu // bsz) * g * tile, ch)],
                        pss)

        for par, (_, rows_v, _, ss) in enumerate(bufs):
            @pl.when(u0 + par < u1)
            def _():
                drain_scatter(rows_v, ss)

    return k(table, idx2d)


def kernel(vertices, reverse_map):
    bsz, n, d = vertices.shape
    table = vertices.reshape(bsz * n, d)
    idx2d = reverse_map.astype(jnp.int32)
    out = _sc_gather(table, idx2d, bsz, n, d, _NBUF, _GTILES)
    return out.reshape(bsz, n, d)


# speedup vs baseline: 1.0035x; 1.0035x over previous
import functools

import jax
import jax.numpy as jnp
from jax import lax
from jax.experimental import pallas as pl
from jax.experimental.pallas import tpu as pltpu
from jax.experimental.pallas import tpu_sc as plsc

_NBUF = 4
_CH = 160
_TILE = 128


@functools.partial(jax.jit, static_argnames=("bsz", "n", "d", "nb", "ch"))
def _sc_gather(table, idx2d, bsz, n, d, nb, ch):
    tile = _TILE
    mesh = plsc.VectorSubcoreMesh(core_axis_name="c", subcore_axis_name="s")
    info = plsc.get_sparse_core_info()
    nw = info.num_cores * info.num_subcores

    main_ranges = (n - ch) // ch if n % tile else n // ch
    spec_w = n - main_ranges * ch
    units = bsz * main_ranges
    u_per_w = -(-units // nw)
    span = (-(-(u_per_w + bsz - 1) // bsz)) * ch + tile - 1
    w_cols = -(-span // tile) * tile
    n_floor = n - n % tile
    assert n % ch == 0 and ch % 16 == 0 and w_cols <= n_floor
    assert main_ranges * ch <= n_floor and spec_w % 16 == 0 and bsz <= nw

    @functools.partial(
        pl.kernel,
        out_type=jax.ShapeDtypeStruct((bsz * n, d), table.dtype),
        mesh=mesh,
        scratch_types=(
            [pltpu.VMEM((bsz, w_cols), jnp.int32)]
            + ([pltpu.VMEM((bsz, spec_w), jnp.int32)] if spec_w else [])
            + [pltpu.VMEM((ch,), jnp.int32) for _ in range(nb)]
            + [pltpu.VMEM((ch, d), table.dtype) for _ in range(nb)]
            + [pltpu.SemaphoreType.DMA for _ in range(2 * nb)]
        ),
    )
    def k(table_hbm, idx_hbm, out_hbm, wdw_v, *scr):
        if spec_w:
            spec_v, scr = scr[0], scr[1:]
        lsts, rows = scr[0:nb], scr[nb:2 * nb]
        semg, sems = scr[2 * nb:3 * nb], scr[3 * nb:]
        bufs = tuple(zip(lsts, rows, semg, sems))
        wid = lax.axis_index("s") * info.num_cores + lax.axis_index("c")
        u0 = (wid * units) // nw
        u1 = ((wid + 1) * units) // nw
        c0 = jnp.minimum(((u0 // bsz) * ch // tile) * tile, n_floor - w_cols)

        def drain_scatter(rows_v, sem):
            pltpu.make_async_copy(
                rows_v, out_hbm.at[pl.ds(0, ch)], sem).wait()

        pltpu.sync_copy(
            idx_hbm.at[:, pl.ds(pl.multiple_of(c0, tile), w_cols)], wdw_v)

        if spec_w:
            @pl.when(wid < bsz)
            def _():
                base_col = main_ranges * ch
                pltpu.sync_copy(
                    idx_hbm.at[:, pl.ds(base_col, spec_w)], spec_v)
                lst_v, rows_v, sg, ss = bufs[0]
                b = wid
                for j in range(spec_w // 16):
                    sl = pl.ds(j * 16, 16)
                    lst_v[sl] = spec_v[b, pl.ds(j * 16, 16)] + b * n
                pltpu.async_copy(
                    table_hbm.at[lst_v.at[pl.ds(0, spec_w)]],
                    rows_v.at[pl.ds(0, spec_w)], sg).wait()
                pltpu.sync_copy(
                    rows_v.at[pl.ds(0, spec_w)],
                    out_hbm.at[pl.ds(b * n + base_col, spec_w)])

        kmax = u_per_w + 1
        kmax_r = kmax + (-kmax) % nb

        @pl.loop(0, kmax_r, step=nb)
        def _(i):
            for par in range(nb):
                k_it = i + par
                u = u0 + k_it
                plst_v, prows_v, psg, pss = bufs[(par - 1) % nb]
                lst_v, rows_v, sg, ss = bufs[par]

                @pl.when(u < u1)
                def _():
                    r = u // bsz
                    b = u % bsz
                    col = r * ch - c0
                    boff = b * n
                    for j in range(ch // 16):
                        sl = pl.ds(j * 16, 16)
                        lst_v[sl] = wdw_v[b, pl.ds(col + j * 16, 16)] + boff

                    @pl.when(k_it >= nb)
                    def _():
                        drain_scatter(rows_v, ss)

                    pltpu.async_copy(table_hbm.at[lst_v], rows_v, sg)

                @pl.when((k_it >= 1) & (u0 + k_it - 1 < u1))
                def _():
                    pu = u0 + k_it - 1
                    pltpu.make_async_copy(
                        table_hbm.at[plst_v], prows_v, psg).wait()
                    pltpu.async_copy(
                        prows_v,
                        out_hbm.at[pl.ds((pu % bsz) * n + (pu // bsz) * ch,
                                         ch)],
                        pss)

        for par, (_, rows_v, _, ss) in enumerate(bufs):
            @pl.when(u0 + par < u1)
            def _():
                drain_scatter(rows_v, ss)

    return k(table, idx2d)


def kernel(vertices, reverse_map):
    bsz, n, d = vertices.shape
    table = vertices.reshape(bsz * n, d)
    idx2d = reverse_map.astype(jnp.int32)
    out = _sc_gather(table, idx2d, bsz, n, d, _NBUF, _CH)
    return out.reshape(bsz, n, d)


# speedup vs baseline: 1.0154x; 1.0119x over previous
import functools

import jax
import jax.numpy as jnp
from jax import lax
from jax.experimental import pallas as pl
from jax.experimental.pallas import tpu as pltpu
from jax.experimental.pallas import tpu_sc as plsc

_NBUF = 4


def _pick_chunk(n_rows_per_batch: int, d: int, nb: int) -> int:
    budget = 460_000 // nb
    best = 0
    for ch in range(16, n_rows_per_batch + 1, 16):
        if n_rows_per_batch % ch:
            continue
        if ch * d * 4 + ch * 4 <= budget:
            best = ch
    if best == 0:
        raise ValueError("no valid chunk size")
    return best


@functools.partial(jax.jit, static_argnames=("bsz", "n", "d", "ch", "nb"))
def _sc_gather(table, idx, bsz, n, d, ch, nb):
    nchunks = (bsz * n) // ch
    chunks_per_batch = n // ch
    mesh = plsc.VectorSubcoreMesh(core_axis_name="c", subcore_axis_name="s")
    info = plsc.get_sparse_core_info()
    nw = info.num_cores * info.num_subcores

    @functools.partial(
        pl.kernel,
        out_type=jax.ShapeDtypeStruct((bsz * n, d), table.dtype),
        mesh=mesh,
        scratch_types=(
            [pltpu.VMEM((ch,), jnp.int32) for _ in range(nb)]
            + [pltpu.VMEM((ch, d), table.dtype) for _ in range(nb)]
            + [pltpu.SemaphoreType.DMA for _ in range(3 * nb)]
        ),
    )
    def k(table_hbm, idx_hbm, out_hbm, *scr):
        idxs, rows = scr[0:nb], scr[nb:2 * nb]
        semi, semg, sems = scr[2 * nb:3 * nb], scr[3 * nb:4 * nb], scr[4 * nb:]
        bufs = tuple(zip(idxs, rows, semi, semg, sems))
        wid = lax.axis_index("s") * info.num_cores + lax.axis_index("c")
        iters = (nchunks + nw - 1) // nw

        def drain_scatter(rows_v, sem):
            pltpu.make_async_copy(rows_v, out_hbm.at[pl.ds(0, ch)], sem).wait()

        def idx_src(c):
            return idx_hbm.at[pl.ds(c * ch, ch)]

        for par, (idx_v, _, si, _, _) in enumerate(bufs):
            c0 = wid + par * nw

            @pl.when(c0 < nchunks)
            def _():
                pltpu.async_copy(idx_src(c0), idx_v, si)

        kmax = iters + 1
        kmax_r = kmax + (-kmax) % nb

        @pl.loop(0, kmax_r, step=nb)
        def _(i):
            for par in range(nb):
                k_it = i + par
                c = wid + k_it * nw
                idx_v, rows_v, si, sg, ss = bufs[par]
                pidx_v, prows_v, psi, psg, pss = bufs[(par - 1) % nb]

                @pl.when(c < nchunks)
                def _():
                    pltpu.make_async_copy(
                        idx_hbm.at[pl.ds(0, ch)], idx_v, si).wait()
                    boff = (c // chunks_per_batch) * n

                    @pl.loop(0, ch, step=16)
                    def _(j):
                        sl = pl.ds(j, 16)
                        idx_v[sl] = idx_v[sl] + boff

                    @pl.when(k_it >= nb)
                    def _():
                        drain_scatter(rows_v, ss)

                    pltpu.async_copy(table_hbm.at[idx_v], rows_v, sg)

                @pl.when((k_it >= 1) & (c - nw < nchunks))
                def _():
                    pltpu.make_async_copy(
                        table_hbm.at[pidx_v], prows_v, psg).wait()
                    pltpu.async_copy(
                        prows_v, out_hbm.at[pl.ds((c - nw) * ch, ch)], pss)

                    @pl.when(c + (nb - 1) * nw < nchunks)
                    def _():
                        pltpu.async_copy(
                            idx_src(c + (nb - 1) * nw), pidx_v, psi)

        for par, (_, rows_v, _, _, ss) in enumerate(bufs):
            @pl.when(wid + par * nw < nchunks)
            def _():
                drain_scatter(rows_v, ss)

    return k(table, idx)


def kernel(vertices, reverse_map):
    bsz, n, d = vertices.shape
    ch = _pick_chunk(n, d, _NBUF)
    table = vertices.reshape(bsz * n, d)
    idx = reverse_map.reshape(bsz * n).astype(jnp.int32)
    out = _sc_gather(table, idx, bsz, n, d, ch, _NBUF)
    return out.reshape(bsz, n, d)
